# baseline (speedup 1.0000x reference)
import dataclasses
import functools

import jax
import jax.numpy as jnp
from jax import lax
from jax.experimental import pallas as pl
from jax.experimental.pallas import tpu as pltpu
from jax.experimental.pallas import tpu_sc as plsc

N, H, W, C = 2, 56, 56, 512
G, GC = 8, 64
P = 9
L = H * W
OM = G * 3 * P
NPIX = N * L

NWORK = 32
CP = 16
NCHUNK = NPIX // CP
NITEM = NCHUNK * G
IPW = NITEM // NWORK
NTAP = P * 4
ROWS = NTAP * CP
IDX_MINOR = 96
IDX_MAJOR = ROWS // IDX_MINOR

_F32 = jnp.float32
_I32 = jnp.int32


def _sc_compiler_params():
    cp = pltpu.CompilerParams()
    fields = pltpu.CompilerParams.__dataclass_fields__
    if "needs_layout_passes" in fields:
        cp = dataclasses.replace(cp, needs_layout_passes=False)
    if "use_tc_tiling_on_sc" in fields:
        cp = dataclasses.replace(cp, use_tc_tiling_on_sc=False)
    return cp


def _iota16():
    return lax.iota(_I32, CP)


def _full16(v, dtype=_I32):
    return jnp.full((CP,), v, dtype)


def _floor_i32(v):
    t = v.astype(_I32)
    return jnp.where(v < t.astype(_F32), t - 1, t)


def _sc_body(val_hbm, om_hbm, out_hbm,
             omb0, omb1, idxb0, idxb1, wb0, wb1, rowsb0, rowsb1,
             outb0, outb1, oidx0, oidx1, semo, semg0, semg1, sems):
    wid = lax.axis_index("c") * 16 + lax.axis_index("s")
    base = wid * IPW
    iota = _iota16()

    def _split(item):
        chunk = item // G
        g = item - chunk * G
        return chunk * CP, g

    def om_issue(item, omb):
        p0, g = _split(item)
        pltpu.async_copy(om_hbm.at[pl.ds(p0, CP), g], omb, semo)

    def om_wait(omb):
        pltpu.make_async_copy(om_hbm.at[pl.ds(0, CP), 0], omb, semo).wait()

    def stage_issue(item, omb, idxb, wb, rowsb, semg):
        p0, g = _split(item)
        pix = p0 + iota
        n = pix // L
        l = pix - n * L
        y = l // W
        x = l - y * W
        xf = x.astype(_F32)
        yf = y.astype(_F32)
        rb = n * (L * G) + g

        for t in range(P):
            offx = plsc.load_gather(omb, [iota, _full16(2 * t)])
            offy = plsc.load_gather(omb, [iota, _full16(2 * t + 1)])
            m = plsc.load_gather(omb, [iota, _full16(2 * P + t)])
            fx = xf + float(t // 3 - 1) + offx
            fy = yf + float(t % 3 - 1) + offy
            x0 = _floor_i32(fx)
            y0 = _floor_i32(fy)
            wx1 = fx - x0.astype(_F32)
            wx0 = 1.0 - wx1
            wy1 = fy - y0.astype(_F32)
            wy0 = 1.0 - wy1
            zero = jnp.zeros((CP,), _F32)
            wx0 = jnp.where((x0 >= 0) & (x0 <= W - 1), wx0, zero)
            wx1 = jnp.where((x0 >= -1) & (x0 <= W - 2), wx1, zero)
            wy0 = jnp.where((y0 >= 0) & (y0 <= H - 1), wy0, zero)
            wy1 = jnp.where((y0 >= -1) & (y0 <= H - 2), wy1, zero)
            cx0 = jnp.clip(x0, 0, W - 1)
            cx1 = jnp.clip(x0 + 1, 0, W - 1)
            cy0 = jnp.clip(y0, 0, H - 1)
            cy1 = jnp.clip(y0 + 1, 0, H - 1)
            rx0 = cx0 * G
            rx1 = cx1 * G
            ry0 = cy0 * (W * G)
            ry1 = cy1 * (W * G)
            corners = (
                (ry0 + rx0, wx0 * wy0),
                (ry0 + rx1, wx1 * wy0),
                (ry1 + rx0, wx0 * wy1),
                (ry1 + rx1, wx1 * wy1),
            )
            for ci, (radd, wgt) in enumerate(corners):
                flat = (t * 4 + ci) * CP
                idxb.at[flat // IDX_MINOR, pl.ds(flat % IDX_MINOR, CP)][...] = rb + radd
                wb.at[pl.ds(flat, CP)][...] = wgt * m

        for i in range(IDX_MAJOR):
            pltpu.async_copy(
                val_hbm.at[idxb.at[i]],
                rowsb.at[pl.ds(i * IDX_MINOR, IDX_MINOR)], semg)

    def gather_wait(idxb, rowsb, semg):
        for i in range(IDX_MAJOR):
            pltpu.make_async_copy(
                val_hbm.at[idxb.at[i]],
                rowsb.at[pl.ds(i * IDX_MINOR, IDX_MINOR)], semg).wait()

    def consume(item, k, wb, rowsb, outb, oidx):
        @pl.when(k >= 2)
        def _():
            pltpu.make_async_copy(outb, out_hbm.at[oidx], sems).wait()

        p0, g = _split(item)
        pix = p0 + iota

        @pl.loop(0, CP)
        def _px(p):
            acc_a = [jnp.zeros((CP,), _F32) for _ in range(GC // CP)]
            acc_b = [jnp.zeros((CP,), _F32) for _ in range(GC // CP)]
            for tc in range(NTAP):
                wsp = plsc.load_gather(wb, [_full16(tc * CP) + p])
                row = tc * CP + p
                tgt = acc_a if tc % 2 == 0 else acc_b
                for c in range(GC // CP):
                    v = rowsb[row, pl.ds(c * CP, CP)]
                    tgt[c] = tgt[c] + wsp * v
            for c in range(GC // CP):
                outb.at[p, pl.ds(c * CP, CP)][...] = acc_a[c] + acc_b[c]

        oidx[...] = pix * G + g
        pltpu.async_copy(outb, out_hbm.at[oidx], sems)

    p0, g0 = _split(base)
    pltpu.sync_copy(om_hbm.at[pl.ds(p0, CP), g0], omb0)
    om_issue(base + 1, omb1)
    stage_issue(base, omb0, idxb0, wb0, rowsb0, semg0)

    @pl.loop(0, IPW, step=2)
    def _pair(k):
        i0 = base + k

        om_wait(omb1)

        @pl.when(k + 2 < IPW)
        def _():
            om_issue(i0 + 2, omb0)

        stage_issue(i0 + 1, omb1, idxb1, wb1, rowsb1, semg1)
        gather_wait(idxb0, rowsb0, semg0)
        consume(i0, k, wb0, rowsb0, outb0, oidx0)

        @pl.when(k + 2 < IPW)
        def _():
            om_wait(omb0)

            @pl.when(k + 3 < IPW)
            def _():
                om_issue(i0 + 3, omb1)

            stage_issue(i0 + 2, omb0, idxb0, wb0, rowsb0, semg0)

        gather_wait(idxb1, rowsb1, semg1)
        consume(i0 + 1, k + 1, wb1, rowsb1, outb1, oidx1)

    pltpu.make_async_copy(outb0, out_hbm.at[oidx0], sems).wait()
    pltpu.make_async_copy(outb1, out_hbm.at[oidx1], sems).wait()


@jax.jit
def _dcn_sc(val2d, om2d):
    mesh = plsc.VectorSubcoreMesh(core_axis_name="c", subcore_axis_name="s")
    f = pl.kernel(
        _sc_body,
        out_type=jax.ShapeDtypeStruct((NPIX * G, GC), _F32),
        mesh=mesh,
        scratch_types=[
            pltpu.VMEM((CP, 3 * P), _F32),
            pltpu.VMEM((CP, 3 * P), _F32),
            pltpu.VMEM((IDX_MAJOR, IDX_MINOR), _I32),
            pltpu.VMEM((IDX_MAJOR, IDX_MINOR), _I32),
            pltpu.VMEM((ROWS,), _F32),
            pltpu.VMEM((ROWS,), _F32),
            pltpu.VMEM((ROWS, GC), _F32),
            pltpu.VMEM((ROWS, GC), _F32),
            pltpu.VMEM((CP, GC), _F32),
            pltpu.VMEM((CP, GC), _F32),
            pltpu.VMEM((CP,), _I32),
            pltpu.VMEM((CP,), _I32),
            pltpu.SemaphoreType.DMA,
            pltpu.SemaphoreType.DMA,
            pltpu.SemaphoreType.DMA,
            pltpu.SemaphoreType.DMA,
        ],
        compiler_params=_sc_compiler_params(),
    )
    return f(val2d, om2d)


_MBLK = 392


def _proj_body(x_ref, wv_ref, wom_ref, bv_ref, bom_ref, val_ref, om_ref):
    xb = x_ref[...]
    dn = (((1,), (1,)), ((), ()))
    val_ref[...] = lax.dot_general(
        xb, wv_ref[...], dn, precision=lax.Precision.HIGHEST,
        preferred_element_type=_F32) + bv_ref[...]
    om_ref[...] = lax.dot_general(
        xb, wom_ref[...], dn, precision=lax.Precision.HIGHEST,
        preferred_element_type=_F32) + bom_ref[...]


@jax.jit
def _proj(x2d, w_v, w_om, b_v, b_om):
    grid = (NPIX // _MBLK,)
    return pl.pallas_call(
        _proj_body,
        grid=grid,
        in_specs=[
            pl.BlockSpec((_MBLK, C), lambda i: (i, 0)),
            pl.BlockSpec((C, C), lambda i: (0, 0)),
            pl.BlockSpec((OM, C), lambda i: (0, 0)),
            pl.BlockSpec((1, C), lambda i: (0, 0)),
            pl.BlockSpec((1, OM), lambda i: (0, 0)),
        ],
        out_specs=[
            pl.BlockSpec((_MBLK, C), lambda i: (i, 0)),
            pl.BlockSpec((_MBLK, OM), lambda i: (i, 0)),
        ],
        out_shape=[
            jax.ShapeDtypeStruct((NPIX, C), _F32),
            jax.ShapeDtypeStruct((NPIX, OM), _F32),
        ],
    )(x2d, w_v, w_om, b_v, b_om)


def _out_body(a_ref, w_ref, b_ref, o_ref):
    o_ref[...] = lax.dot_general(
        a_ref[...], w_ref[...], (((1,), (1,)), ((), ())),
        precision=lax.Precision.HIGHEST,
        preferred_element_type=_F32) + b_ref[...]


@jax.jit
def _outproj(agg2d, w_o, b_o):
    return pl.pallas_call(
        _out_body,
        grid=(NPIX // _MBLK,),
        in_specs=[
            pl.BlockSpec((_MBLK, C), lambda i: (i, 0)),
            pl.BlockSpec((C, C), lambda i: (0, 0)),
            pl.BlockSpec((1, C), lambda i: (0, 0)),
        ],
        out_specs=pl.BlockSpec((_MBLK, C), lambda i: (i, 0)),
        out_shape=jax.ShapeDtypeStruct((NPIX, C), _F32),
    )(agg2d, w_o, b_o)


def kernel(x, w_v, b_v, w_om, b_om, w_o, b_o):
    x2 = x.reshape(NPIX, C)
    value, om = _proj(x2, w_v, w_om, b_v.reshape(1, C), b_om.reshape(1, OM))
    agg = _dcn_sc(value.reshape(NPIX * G, GC), om.reshape(NPIX, G, 3 * P))
    out = _outproj(agg.reshape(NPIX, C), w_o, b_o.reshape(1, C))
    return out.reshape(N, L, C)


# speedup vs baseline: 1.1301x; 1.1301x over previous
import dataclasses
import functools

import jax
import jax.numpy as jnp
from jax import lax
from jax.experimental import pallas as pl
from jax.experimental.pallas import tpu as pltpu
from jax.experimental.pallas import tpu_sc as plsc

N, H, W, C = 2, 56, 56, 512
G, GC = 8, 64
P = 9
L = H * W
OM = G * 3 * P
NPIX = N * L

NWORK = 32
CP = 16
NCHUNK = NPIX // CP
NITEM = NCHUNK * G
IPW = NITEM // NWORK
NTAP = P * 4
ROWS = NTAP * CP
IDX_MINOR = 96
IDX_MAJOR = ROWS // IDX_MINOR

_F32 = jnp.float32
_I32 = jnp.int32


def _sc_compiler_params():
    cp = pltpu.CompilerParams()
    fields = pltpu.CompilerParams.__dataclass_fields__
    if "needs_layout_passes" in fields:
        cp = dataclasses.replace(cp, needs_layout_passes=False)
    if "use_tc_tiling_on_sc" in fields:
        cp = dataclasses.replace(cp, use_tc_tiling_on_sc=False)
    return cp


def _iota16():
    return lax.iota(_I32, CP)


def _full16(v, dtype=_I32):
    return jnp.full((CP,), v, dtype)


def _floor_i32(v):
    t = v.astype(_I32)
    return jnp.where(v < t.astype(_F32), t - 1, t)


def _sc_body(val_hbm, om_hbm, out_hbm,
             omb0, omb1, idxb0, idxb1, wb0, wb1, rowsb0, rowsb1,
             outb0, outb1, oidx0, oidx1, semo, semg0, semg1, sems):
    wid = lax.axis_index("c") * 16 + lax.axis_index("s")
    base = wid * IPW
    iota = _iota16()

    def _split(item):
        chunk = item // G
        g = item - chunk * G
        return chunk * CP, g

    def om_issue(item, omb):
        p0, g = _split(item)
        pltpu.async_copy(om_hbm.at[pl.ds(p0, CP), g], omb, semo)

    def om_wait(omb):
        pltpu.make_async_copy(om_hbm.at[pl.ds(0, CP), 0], omb, semo).wait()

    def stage_issue(item, omb, idxb, wb, rowsb, semg):
        p0, g = _split(item)
        pix = p0 + iota
        n = pix // L
        l = pix - n * L
        y = l // W
        x = l - y * W
        xf = x.astype(_F32)
        yf = y.astype(_F32)
        rb = n * (L * G) + g

        for t in range(P):
            offx = plsc.load_gather(omb, [iota, _full16(2 * t)])
            offy = plsc.load_gather(omb, [iota, _full16(2 * t + 1)])
            m = plsc.load_gather(omb, [iota, _full16(2 * P + t)])
            fx = xf + float(t // 3 - 1) + offx
            fy = yf + float(t % 3 - 1) + offy
            x0 = _floor_i32(fx)
            y0 = _floor_i32(fy)
            wx1 = fx - x0.astype(_F32)
            wx0 = 1.0 - wx1
            wy1 = fy - y0.astype(_F32)
            wy0 = 1.0 - wy1
            zero = jnp.zeros((CP,), _F32)
            wx0 = jnp.where((x0 >= 0) & (x0 <= W - 1), wx0, zero)
            wx1 = jnp.where((x0 >= -1) & (x0 <= W - 2), wx1, zero)
            wy0 = jnp.where((y0 >= 0) & (y0 <= H - 1), wy0, zero)
            wy1 = jnp.where((y0 >= -1) & (y0 <= H - 2), wy1, zero)
            cx0 = jnp.clip(x0, 0, W - 1)
            cx1 = jnp.clip(x0 + 1, 0, W - 1)
            cy0 = jnp.clip(y0, 0, H - 1)
            cy1 = jnp.clip(y0 + 1, 0, H - 1)
            rx0 = cx0 * G
            rx1 = cx1 * G
            ry0 = cy0 * (W * G)
            ry1 = cy1 * (W * G)
            corners = (
                (ry0 + rx0, wx0 * wy0),
                (ry0 + rx1, wx1 * wy0),
                (ry1 + rx0, wx0 * wy1),
                (ry1 + rx1, wx1 * wy1),
            )
            for ci, (radd, wgt) in enumerate(corners):
                flat = (t * 4 + ci) * CP
                idxb.at[flat // IDX_MINOR, pl.ds(flat % IDX_MINOR, CP)][...] = rb + radd
                wb.at[pl.ds(flat, CP)][...] = wgt * m

        for i in range(IDX_MAJOR):
            pltpu.async_copy(
                val_hbm.at[idxb.at[i]],
                rowsb.at[pl.ds(i * IDX_MINOR, IDX_MINOR)], semg)

    def gather_wait(idxb, rowsb, semg):
        for i in range(IDX_MAJOR):
            pltpu.make_async_copy(
                val_hbm.at[idxb.at[i]],
                rowsb.at[pl.ds(i * IDX_MINOR, IDX_MINOR)], semg).wait()

    def consume(item, k, wb, rowsb, outb, oidx):
        @pl.when(k >= 2)
        def _():
            pltpu.make_async_copy(outb, out_hbm.at[oidx], sems).wait()

        p0, g = _split(item)
        pix = p0 + iota

        @pl.loop(0, CP)
        def _px(p):
            acc_a = [jnp.zeros((CP,), _F32) for _ in range(GC // CP)]
            acc_b = [jnp.zeros((CP,), _F32) for _ in range(GC // CP)]
            for tc in range(NTAP):
                wsp = plsc.load_gather(wb, [_full16(tc * CP) + p])
                row = tc * CP + p
                tgt = acc_a if tc % 2 == 0 else acc_b
                for h in range(2):
                    v2 = rowsb[row, pl.ds(h * 2 * CP, 2 * CP)]
                    ev, od = plsc.unpack(v2, format=plsc.PackFormat.INTERLEAVED)
                    tgt[2 * h] = tgt[2 * h] + wsp * ev
                    tgt[2 * h + 1] = tgt[2 * h + 1] + wsp * od
            for c in range(GC // CP):
                outb.at[p, pl.ds(c * CP, CP)][...] = acc_a[c] + acc_b[c]

        oidx[...] = pix * G + g
        pltpu.async_copy(outb, out_hbm.at[oidx], sems)

    p0, g0 = _split(base)
    pltpu.sync_copy(om_hbm.at[pl.ds(p0, CP), g0], omb0)
    om_issue(base + 1, omb1)
    stage_issue(base, omb0, idxb0, wb0, rowsb0, semg0)

    @pl.loop(0, IPW, step=2)
    def _pair(k):
        i0 = base + k

        om_wait(omb1)

        @pl.when(k + 2 < IPW)
        def _():
            om_issue(i0 + 2, omb0)

        stage_issue(i0 + 1, omb1, idxb1, wb1, rowsb1, semg1)
        gather_wait(idxb0, rowsb0, semg0)
        consume(i0, k, wb0, rowsb0, outb0, oidx0)

        @pl.when(k + 2 < IPW)
        def _():
            om_wait(omb0)

            @pl.when(k + 3 < IPW)
            def _():
                om_issue(i0 + 3, omb1)

            stage_issue(i0 + 2, omb0, idxb0, wb0, rowsb0, semg0)

        gather_wait(idxb1, rowsb1, semg1)
        consume(i0 + 1, k + 1, wb1, rowsb1, outb1, oidx1)

    pltpu.make_async_copy(outb0, out_hbm.at[oidx0], sems).wait()
    pltpu.make_async_copy(outb1, out_hbm.at[oidx1], sems).wait()


@jax.jit
def _dcn_sc(val2d, om2d):
    mesh = plsc.VectorSubcoreMesh(core_axis_name="c", subcore_axis_name="s")
    f = pl.kernel(
        _sc_body,
        out_type=jax.ShapeDtypeStruct((NPIX * G, GC), _F32),
        mesh=mesh,
        scratch_types=[
            pltpu.VMEM((CP, 3 * P), _F32),
            pltpu.VMEM((CP, 3 * P), _F32),
            pltpu.VMEM((IDX_MAJOR, IDX_MINOR), _I32),
            pltpu.VMEM((IDX_MAJOR, IDX_MINOR), _I32),
            pltpu.VMEM((ROWS,), _F32),
            pltpu.VMEM((ROWS,), _F32),
            pltpu.VMEM((ROWS, GC), jnp.bfloat16),
            pltpu.VMEM((ROWS, GC), jnp.bfloat16),
            pltpu.VMEM((CP, GC), _F32),
            pltpu.VMEM((CP, GC), _F32),
            pltpu.VMEM((CP,), _I32),
            pltpu.VMEM((CP,), _I32),
            pltpu.SemaphoreType.DMA,
            pltpu.SemaphoreType.DMA,
            pltpu.SemaphoreType.DMA,
            pltpu.SemaphoreType.DMA,
        ],
        compiler_params=_sc_compiler_params(),
    )
    return f(val2d, om2d)


_MBLK = 392


def _proj_body(x_ref, wv_ref, wom_ref, bv_ref, bom_ref, val_ref, om_ref):
    xb = x_ref[...]
    dn = (((1,), (1,)), ((), ()))
    val_ref[...] = (lax.dot_general(
        xb, wv_ref[...], dn, precision=lax.Precision.HIGHEST,
        preferred_element_type=_F32) + bv_ref[...]).astype(jnp.bfloat16)
    om_ref[...] = lax.dot_general(
        xb, wom_ref[...], dn, precision=lax.Precision.HIGHEST,
        preferred_element_type=_F32) + bom_ref[...]


@jax.jit
def _proj(x2d, w_v, w_om, b_v, b_om):
    grid = (NPIX // _MBLK,)
    return pl.pallas_call(
        _proj_body,
        grid=grid,
        in_specs=[
            pl.BlockSpec((_MBLK, C), lambda i: (i, 0)),
            pl.BlockSpec((C, C), lambda i: (0, 0)),
            pl.BlockSpec((OM, C), lambda i: (0, 0)),
            pl.BlockSpec((1, C), lambda i: (0, 0)),
            pl.BlockSpec((1, OM), lambda i: (0, 0)),
        ],
        out_specs=[
            pl.BlockSpec((_MBLK, C), lambda i: (i, 0)),
            pl.BlockSpec((_MBLK, OM), lambda i: (i, 0)),
        ],
        out_shape=[
            jax.ShapeDtypeStruct((NPIX, C), jnp.bfloat16),
            jax.ShapeDtypeStruct((NPIX, OM), _F32),
        ],
    )(x2d, w_v, w_om, b_v, b_om)


def _out_body(a_ref, w_ref, b_ref, o_ref):
    o_ref[...] = lax.dot_general(
        a_ref[...], w_ref[...], (((1,), (1,)), ((), ())),
        precision=lax.Precision.HIGHEST,
        preferred_element_type=_F32) + b_ref[...]


@jax.jit
def _outproj(agg2d, w_o, b_o):
    return pl.pallas_call(
        _out_body,
        grid=(NPIX // _MBLK,),
        in_specs=[
            pl.BlockSpec((_MBLK, C), lambda i: (i, 0)),
            pl.BlockSpec((C, C), lambda i: (0, 0)),
            pl.BlockSpec((1, C), lambda i: (0, 0)),
        ],
        out_specs=pl.BlockSpec((_MBLK, C), lambda i: (i, 0)),
        out_shape=jax.ShapeDtypeStruct((NPIX, C), _F32),
    )(agg2d, w_o, b_o)


_CPERM = jnp.asarray(
    [(g * GC + (b // 2) * 32 + (b % 2) + 2 * j)
     for g in range(G) for b in range(4) for j in range(CP)], jnp.int32)


def kernel(x, w_v, b_v, w_om, b_om, w_o, b_o):
    x2 = x.reshape(NPIX, C)
    value, om = _proj(x2, w_v, w_om, b_v.reshape(1, C), b_om.reshape(1, OM))
    agg = _dcn_sc(value.reshape(NPIX * G, GC), om.reshape(NPIX, G, 3 * P))
    out = _outproj(agg.reshape(NPIX, C), w_o[:, _CPERM], b_o.reshape(1, C))
    return out.reshape(N, L, C)


# speedup vs baseline: 1.2292x; 1.0877x over previous
import dataclasses
import functools

import jax
import jax.numpy as jnp
from jax import lax
from jax.experimental import pallas as pl
from jax.experimental.pallas import tpu as pltpu
from jax.experimental.pallas import tpu_sc as plsc

N, H, W, C = 2, 56, 56, 512
G, GC = 8, 64
P = 9
L = H * W
OM = G * 3 * P
NPIX = N * L

NWORK = 32
CP = 16
NCHUNK = NPIX // CP
NITEM = NCHUNK * G
IPW = NITEM // NWORK
NTAP = P * 4
ROWS = NTAP * CP
IDX_MINOR = 96
IDX_MAJOR = ROWS // IDX_MINOR

_F32 = jnp.float32
_I32 = jnp.int32


def _sc_compiler_params():
    cp = pltpu.CompilerParams()
    fields = pltpu.CompilerParams.__dataclass_fields__
    if "needs_layout_passes" in fields:
        cp = dataclasses.replace(cp, needs_layout_passes=False)
    if "use_tc_tiling_on_sc" in fields:
        cp = dataclasses.replace(cp, use_tc_tiling_on_sc=False)
    return cp


def _iota16():
    return lax.iota(_I32, CP)


def _full16(v, dtype=_I32):
    return jnp.full((CP,), v, dtype)


def _floor_i32(v):
    t = v.astype(_I32)
    return jnp.where(v < t.astype(_F32), t - 1, t)


def _sc_body(val_hbm, om_hbm, out_hbm,
             omb0, omb1, idxb0, idxb1, wb0, wb1, rowsb0, rowsb1,
             outb0, outb1, oidx0, oidx1, semo, semg0, semg1, sems):
    wid = lax.axis_index("c") * 16 + lax.axis_index("s")
    base = wid * IPW
    iota = _iota16()

    def _split(item):
        chunk = item // G
        g = item - chunk * G
        return chunk * CP, g

    def om_issue(item, omb):
        p0, g = _split(item)
        pltpu.async_copy(om_hbm.at[pl.ds(p0, CP), g], omb, semo)

    def om_wait(omb):
        pltpu.make_async_copy(om_hbm.at[pl.ds(0, CP), 0], omb, semo).wait()

    def stage_issue(item, omb, idxb, wb, rowsb, semg):
        p0, g = _split(item)
        pix = p0 + iota
        n = pix // L
        l = pix - n * L
        y = l // W
        x = l - y * W
        xf = x.astype(_F32)
        yf = y.astype(_F32)
        rb = n * (L * G) + g

        for t in range(P):
            offx = plsc.load_gather(omb, [iota, _full16(2 * t)])
            offy = plsc.load_gather(omb, [iota, _full16(2 * t + 1)])
            m = plsc.load_gather(omb, [iota, _full16(2 * P + t)])
            fx = xf + float(t // 3 - 1) + offx
            fy = yf + float(t % 3 - 1) + offy
            x0 = _floor_i32(fx)
            y0 = _floor_i32(fy)
            wx1 = fx - x0.astype(_F32)
            wx0 = 1.0 - wx1
            wy1 = fy - y0.astype(_F32)
            wy0 = 1.0 - wy1
            zero = jnp.zeros((CP,), _F32)
            wx0 = jnp.where((x0 >= 0) & (x0 <= W - 1), wx0, zero)
            wx1 = jnp.where((x0 >= -1) & (x0 <= W - 2), wx1, zero)
            wy0 = jnp.where((y0 >= 0) & (y0 <= H - 1), wy0, zero)
            wy1 = jnp.where((y0 >= -1) & (y0 <= H - 2), wy1, zero)
            cx0 = jnp.clip(x0, 0, W - 1)
            cx1 = jnp.clip(x0 + 1, 0, W - 1)
            cy0 = jnp.clip(y0, 0, H - 1)
            cy1 = jnp.clip(y0 + 1, 0, H - 1)
            rx0 = cx0 * G
            rx1 = cx1 * G
            ry0 = cy0 * (W * G)
            ry1 = cy1 * (W * G)
            corners = (
                (ry0 + rx0, wx0 * wy0),
                (ry0 + rx1, wx1 * wy0),
                (ry1 + rx0, wx0 * wy1),
                (ry1 + rx1, wx1 * wy1),
            )
            for ci, (radd, wgt) in enumerate(corners):
                flat = (t * 4 + ci) * CP
                idxb.at[flat // IDX_MINOR, pl.ds(flat % IDX_MINOR, CP)][...] = rb + radd
                wb.at[pl.ds(flat, CP)][...] = wgt * m

        for i in range(IDX_MAJOR):
            pltpu.async_copy(
                val_hbm.at[idxb.at[i]],
                rowsb.at[pl.ds(i * IDX_MINOR, IDX_MINOR)], semg)

    def gather_wait(idxb, rowsb, semg):
        for i in range(IDX_MAJOR):
            pltpu.make_async_copy(
                val_hbm.at[idxb.at[i]],
                rowsb.at[pl.ds(i * IDX_MINOR, IDX_MINOR)], semg).wait()

    def consume(item, k, wb, rowsb, outb, oidx):
        @pl.when(k >= 2)
        def _():
            pltpu.make_async_copy(outb, out_hbm.at[oidx], sems).wait()

        p0, g = _split(item)
        pix = p0 + iota

        @pl.loop(0, CP)
        def _px(p):
            acc_a = [jnp.zeros((CP,), _F32) for _ in range(GC // CP)]
            acc_b = [jnp.zeros((CP,), _F32) for _ in range(GC // CP)]
            for tc in range(NTAP):
                wsp = plsc.load_gather(wb, [_full16(tc * CP) + p])
                row = tc * CP + p
                tgt = acc_a if tc % 2 == 0 else acc_b
                for h in range(2):
                    v2 = rowsb[row, pl.ds(h * 2 * CP, 2 * CP)]
                    ev, od = plsc.unpack(v2, format=plsc.PackFormat.INTERLEAVED)
                    tgt[2 * h] = tgt[2 * h] + wsp * ev
                    tgt[2 * h + 1] = tgt[2 * h + 1] + wsp * od
            for c in range(GC // CP):
                outb.at[p, pl.ds(c * CP, CP)][...] = acc_a[c] + acc_b[c]

        oidx[...] = pix * G + g
        pltpu.async_copy(outb, out_hbm.at[oidx], sems)

    p0, g0 = _split(base)
    pltpu.sync_copy(om_hbm.at[pl.ds(p0, CP), g0], omb0)
    om_issue(base + 1, omb1)
    stage_issue(base, omb0, idxb0, wb0, rowsb0, semg0)

    @pl.loop(0, IPW, step=2)
    def _pair(k):
        i0 = base + k

        om_wait(omb1)

        @pl.when(k + 2 < IPW)
        def _():
            om_issue(i0 + 2, omb0)

        stage_issue(i0 + 1, omb1, idxb1, wb1, rowsb1, semg1)
        gather_wait(idxb0, rowsb0, semg0)
        consume(i0, k, wb0, rowsb0, outb0, oidx0)

        @pl.when(k + 2 < IPW)
        def _():
            om_wait(omb0)

            @pl.when(k + 3 < IPW)
            def _():
                om_issue(i0 + 3, omb1)

            stage_issue(i0 + 2, omb0, idxb0, wb0, rowsb0, semg0)

        gather_wait(idxb1, rowsb1, semg1)
        consume(i0 + 1, k + 1, wb1, rowsb1, outb1, oidx1)

    pltpu.make_async_copy(outb0, out_hbm.at[oidx0], sems).wait()
    pltpu.make_async_copy(outb1, out_hbm.at[oidx1], sems).wait()


@jax.jit
def _dcn_sc(val2d, om2d):
    mesh = plsc.VectorSubcoreMesh(core_axis_name="c", subcore_axis_name="s")
    f = pl.kernel(
        _sc_body,
        out_type=jax.ShapeDtypeStruct((NPIX * G, GC), _F32),
        mesh=mesh,
        scratch_types=[
            pltpu.VMEM((CP, 3 * P), _F32),
            pltpu.VMEM((CP, 3 * P), _F32),
            pltpu.VMEM((IDX_MAJOR, IDX_MINOR), _I32),
            pltpu.VMEM((IDX_MAJOR, IDX_MINOR), _I32),
            pltpu.VMEM((ROWS,), _F32),
            pltpu.VMEM((ROWS,), _F32),
            pltpu.VMEM((ROWS, GC), jnp.bfloat16),
            pltpu.VMEM((ROWS, GC), jnp.bfloat16),
            pltpu.VMEM((CP, GC), _F32),
            pltpu.VMEM((CP, GC), _F32),
            pltpu.VMEM((CP,), _I32),
            pltpu.VMEM((CP,), _I32),
            pltpu.SemaphoreType.DMA,
            pltpu.SemaphoreType.DMA,
            pltpu.SemaphoreType.DMA,
            pltpu.SemaphoreType.DMA,
        ],
        compiler_params=_sc_compiler_params(),
    )
    return f(val2d, om2d)


_MBLK = 392


def _proj_body(x_ref, wv_ref, wom_ref, bv_ref, bom_ref, val_ref, om_ref):
    xb = x_ref[...]
    dn = (((1,), (1,)), ((), ()))
    xb16 = xb.astype(jnp.bfloat16)
    val_ref[...] = (lax.dot_general(
        xb16, wv_ref[...].astype(jnp.bfloat16), dn,
        preferred_element_type=_F32) + bv_ref[...]).astype(jnp.bfloat16)
    om_ref[...] = lax.dot_general(
        xb, wom_ref[...], dn,
        preferred_element_type=_F32) + bom_ref[...]


@jax.jit
def _proj(x2d, w_v, w_om, b_v, b_om):
    grid = (NPIX // _MBLK,)
    return pl.pallas_call(
        _proj_body,
        grid=grid,
        in_specs=[
            pl.BlockSpec((_MBLK, C), lambda i: (i, 0)),
            pl.BlockSpec((C, C), lambda i: (0, 0)),
            pl.BlockSpec((OM, C), lambda i: (0, 0)),
            pl.BlockSpec((1, C), lambda i: (0, 0)),
            pl.BlockSpec((1, OM), lambda i: (0, 0)),
        ],
        out_specs=[
            pl.BlockSpec((_MBLK, C), lambda i: (i, 0)),
            pl.BlockSpec((_MBLK, OM), lambda i: (i, 0)),
        ],
        out_shape=[
            jax.ShapeDtypeStruct((NPIX, C), jnp.bfloat16),
            jax.ShapeDtypeStruct((NPIX, OM), _F32),
        ],
    )(x2d, w_v, w_om, b_v, b_om)


def _out_body(a_ref, w_ref, b_ref, o_ref):
    o_ref[...] = lax.dot_general(
        a_ref[...].astype(jnp.bfloat16), w_ref[...].astype(jnp.bfloat16),
        (((1,), (1,)), ((), ())),
        preferred_element_type=_F32) + b_ref[...]


@jax.jit
def _outproj(agg2d, w_o, b_o):
    return pl.pallas_call(
        _out_body,
        grid=(NPIX // _MBLK,),
        in_specs=[
            pl.BlockSpec((_MBLK, C), lambda i: (i, 0)),
            pl.BlockSpec((C, C), lambda i: (0, 0)),
            pl.BlockSpec((1, C), lambda i: (0, 0)),
        ],
        out_specs=pl.BlockSpec((_MBLK, C), lambda i: (i, 0)),
        out_shape=jax.ShapeDtypeStruct((NPIX, C), _F32),
    )(agg2d, w_o, b_o)


_CPERM = jnp.asarray(
    [(g * GC + (b // 2) * 32 + (b % 2) + 2 * j)
     for g in range(G) for b in range(4) for j in range(CP)], jnp.int32)


def kernel(x, w_v, b_v, w_om, b_om, w_o, b_o):
    x2 = x.reshape(NPIX, C)
    value, om = _proj(x2, w_v, w_om, b_v.reshape(1, C), b_om.reshape(1, OM))
    agg = _dcn_sc(value.reshape(NPIX * G, GC), om.reshape(NPIX, G, 3 * P))
    out = _outproj(agg.reshape(NPIX, C), w_o[:, _CPERM], b_o.reshape(1, C))
    return out.reshape(N, L, C)


# speedup vs baseline: 1.3253x; 1.0781x over previous
import dataclasses
import functools

import jax
import jax.numpy as jnp
from jax import lax
from jax.experimental import pallas as pl
from jax.experimental.pallas import tpu as pltpu
from jax.experimental.pallas import tpu_sc as plsc

N, H, W, C = 2, 56, 56, 512
G, GC = 8, 64
P = 9
L = H * W
OM = G * 3 * P
NPIX = N * L

NWORK = 32
CP = 16
NCHUNK = NPIX // CP
NITEM = NCHUNK * G
IPW = NITEM // NWORK
NTAP = P * 4
ROWS = NTAP * CP
IDX_MINOR = 96
IDX_MAJOR = ROWS // IDX_MINOR

_F32 = jnp.float32
_I32 = jnp.int32


def _sc_compiler_params():
    cp = pltpu.CompilerParams()
    fields = pltpu.CompilerParams.__dataclass_fields__
    if "needs_layout_passes" in fields:
        cp = dataclasses.replace(cp, needs_layout_passes=False)
    if "use_tc_tiling_on_sc" in fields:
        cp = dataclasses.replace(cp, use_tc_tiling_on_sc=False)
    return cp


def _iota16():
    return lax.iota(_I32, CP)


def _full16(v, dtype=_I32):
    return jnp.full((CP,), v, dtype)


def _floor_i32(v):
    t = v.astype(_I32)
    return jnp.where(v < t.astype(_F32), t - 1, t)


def _sc_body(val_hbm, om_hbm, out_hbm,
             omb0, omb1, idxb0, idxb1, wb0, wb1, rowsb0, rowsb1,
             outb0, outb1, oidx0, oidx1, semo, semg0, semg1, sems):
    wid = lax.axis_index("c") * 16 + lax.axis_index("s")
    base = wid * IPW
    iota = _iota16()

    def _split(item):
        chunk = item // G
        g = item - chunk * G
        return chunk * CP, g

    valt = val_hbm
    outt = out_hbm

    def om_issue(item, omb):
        p0, g = _split(item)
        pltpu.async_copy(
            om_hbm.at[pl.ds(g * (3 * P), 3 * P), pl.ds(p0, CP)], omb, semo)

    def om_wait(omb):
        pltpu.make_async_copy(
            om_hbm.at[pl.ds(0, 3 * P), pl.ds(0, CP)], omb, semo).wait()

    def stage_issue(item, omb, idxb, wb, rowsb, semg):
        p0, g = _split(item)
        pix = p0 + iota
        n = pix // L
        l = pix - n * L
        y = l // W
        x = l - y * W
        xf = x.astype(_F32)
        yf = y.astype(_F32)
        rb = n * (L * G) + g

        for t in range(P):
            offx = omb[2 * t, :]
            offy = omb[2 * t + 1, :]
            m = omb[2 * P + t, :]
            fx = xf + float(t // 3 - 1) + offx
            fy = yf + float(t % 3 - 1) + offy
            x0 = _floor_i32(fx)
            y0 = _floor_i32(fy)
            wx1 = fx - x0.astype(_F32)
            wx0 = 1.0 - wx1
            wy1 = fy - y0.astype(_F32)
            wy0 = 1.0 - wy1
            zero = jnp.zeros((CP,), _F32)
            wx0 = jnp.where((x0 >= 0) & (x0 <= W - 1), wx0, zero)
            wx1 = jnp.where((x0 >= -1) & (x0 <= W - 2), wx1, zero)
            wy0 = jnp.where((y0 >= 0) & (y0 <= H - 1), wy0, zero)
            wy1 = jnp.where((y0 >= -1) & (y0 <= H - 2), wy1, zero)
            cx0 = jnp.clip(x0, 0, W - 1)
            cx1 = jnp.clip(x0 + 1, 0, W - 1)
            cy0 = jnp.clip(y0, 0, H - 1)
            cy1 = jnp.clip(y0 + 1, 0, H - 1)
            rx0 = cx0 * G
            rx1 = cx1 * G
            ry0 = cy0 * (W * G)
            ry1 = cy1 * (W * G)
            corners = (
                (ry0 + rx0, wx0 * wy0),
                (ry0 + rx1, wx1 * wy0),
                (ry1 + rx0, wx0 * wy1),
                (ry1 + rx1, wx1 * wy1),
            )
            for ci, (radd, wgt) in enumerate(corners):
                flat = (t * 4 + ci) * CP
                idxb.at[flat // IDX_MINOR, pl.ds(flat % IDX_MINOR, CP)][...] = rb + radd
                wb.at[pl.ds(flat, CP)][...] = wgt * m

        for i in range(IDX_MAJOR):
            pltpu.async_copy(
                valt.at[idxb.at[i]],
                rowsb.at[pl.ds(i * IDX_MINOR, IDX_MINOR)], semg)

    def gather_wait(idxb, rowsb, semg):
        for i in range(IDX_MAJOR):
            pltpu.make_async_copy(
                valt.at[idxb.at[i]],
                rowsb.at[pl.ds(i * IDX_MINOR, IDX_MINOR)], semg).wait()

    def consume(item, k, wb, rowsb, outb, oidx):
        @pl.when(k >= 2)
        def _():
            pltpu.make_async_copy(outb, outt.at[oidx], sems).wait()

        p0, g = _split(item)
        pix = p0 + iota

        @pl.loop(0, CP)
        def _px(p):
            acc_a = [jnp.zeros((CP,), _F32) for _ in range(GC // CP)]
            acc_b = [jnp.zeros((CP,), _F32) for _ in range(GC // CP)]
            for tc in range(NTAP):
                wsp = plsc.load_gather(wb, [_full16(tc * CP) + p])
                row = tc * CP + p
                tgt = acc_a if tc % 2 == 0 else acc_b
                for h in range(2):
                    v2 = rowsb[row, pl.ds(h * 2 * CP, 2 * CP)]
                    ev, od = plsc.unpack(v2, format=plsc.PackFormat.INTERLEAVED)
                    tgt[2 * h] = tgt[2 * h] + wsp * ev
                    tgt[2 * h + 1] = tgt[2 * h + 1] + wsp * od
            for c in range(GC // CP):
                outb.at[p, pl.ds(c * CP, CP)][...] = acc_a[c] + acc_b[c]

        oidx[...] = pix * G + g
        pltpu.async_copy(outb, outt.at[oidx], sems)

    p0, g0 = _split(base)
    pltpu.sync_copy(om_hbm.at[pl.ds(g0 * (3 * P), 3 * P), pl.ds(p0, CP)], omb0)
    om_issue(base + 1, omb1)
    stage_issue(base, omb0, idxb0, wb0, rowsb0, semg0)

    @pl.loop(0, IPW, step=2)
    def _pair(k):
        i0 = base + k

        om_wait(omb1)

        @pl.when(k + 2 < IPW)
        def _():
            om_issue(i0 + 2, omb0)

        stage_issue(i0 + 1, omb1, idxb1, wb1, rowsb1, semg1)
        gather_wait(idxb0, rowsb0, semg0)
        consume(i0, k, wb0, rowsb0, outb0, oidx0)

        @pl.when(k + 2 < IPW)
        def _():
            om_wait(omb0)

            @pl.when(k + 3 < IPW)
            def _():
                om_issue(i0 + 3, omb1)

            stage_issue(i0 + 2, omb0, idxb0, wb0, rowsb0, semg0)

        gather_wait(idxb1, rowsb1, semg1)
        consume(i0 + 1, k + 1, wb1, rowsb1, outb1, oidx1)

    pltpu.make_async_copy(outb0, outt.at[oidx0], sems).wait()
    pltpu.make_async_copy(outb1, outt.at[oidx1], sems).wait()


@jax.jit
def _dcn_sc(val2d, om2d):
    mesh = plsc.VectorSubcoreMesh(core_axis_name="c", subcore_axis_name="s")
    f = pl.kernel(
        _sc_body,
        out_type=jax.ShapeDtypeStruct((NPIX * G, GC), _F32),
        mesh=mesh,
        scratch_types=[
            pltpu.VMEM((3 * P, CP), _F32),
            pltpu.VMEM((3 * P, CP), _F32),
            pltpu.VMEM((IDX_MAJOR, IDX_MINOR), _I32),
            pltpu.VMEM((IDX_MAJOR, IDX_MINOR), _I32),
            pltpu.VMEM((ROWS,), _F32),
            pltpu.VMEM((ROWS,), _F32),
            pltpu.VMEM((ROWS, GC), jnp.bfloat16),
            pltpu.VMEM((ROWS, GC), jnp.bfloat16),
            pltpu.VMEM((CP, GC), _F32),
            pltpu.VMEM((CP, GC), _F32),
            pltpu.VMEM((CP,), _I32),
            pltpu.VMEM((CP,), _I32),
            pltpu.SemaphoreType.DMA,
            pltpu.SemaphoreType.DMA,
            pltpu.SemaphoreType.DMA,
            pltpu.SemaphoreType.DMA,
        ],
        compiler_params=_sc_compiler_params(),
    )
    return f(val2d, om2d)


_MBLK = 128


def _proj_body(x_ref, wv_ref, wom_ref, bv_ref, bom_ref, val_ref, om_ref):
    xb = x_ref[...]
    dn = (((1,), (1,)), ((), ()))
    xb16 = xb.astype(jnp.bfloat16)
    val_ref[...] = (lax.dot_general(
        xb16, wv_ref[...].astype(jnp.bfloat16), dn,
        preferred_element_type=_F32) + bv_ref[...]).astype(jnp.bfloat16)
    om_ref[...] = lax.dot_general(
        wom_ref[...], xb, dn,
        preferred_element_type=_F32) + bom_ref[...]


@jax.jit
def _proj(x2d, w_v, w_om, b_v, b_om):
    grid = (NPIX // _MBLK,)
    return pl.pallas_call(
        _proj_body,
        grid=grid,
        in_specs=[
            pl.BlockSpec((_MBLK, C), lambda i: (i, 0)),
            pl.BlockSpec((C, C), lambda i: (0, 0)),
            pl.BlockSpec((OM, C), lambda i: (0, 0)),
            pl.BlockSpec((1, C), lambda i: (0, 0)),
            pl.BlockSpec((OM, 1), lambda i: (0, 0)),
        ],
        out_specs=[
            pl.BlockSpec((_MBLK, C), lambda i: (i, 0)),
            pl.BlockSpec((OM, _MBLK), lambda i: (0, i)),
        ],
        out_shape=[
            jax.ShapeDtypeStruct((NPIX, C), jnp.bfloat16),
            jax.ShapeDtypeStruct((OM, NPIX), _F32),
        ],
    )(x2d, w_v, w_om, b_v, b_om)


def _out_body(a_ref, w_ref, b_ref, o_ref):
    o_ref[...] = lax.dot_general(
        a_ref[...].astype(jnp.bfloat16), w_ref[...].astype(jnp.bfloat16),
        (((1,), (1,)), ((), ())),
        preferred_element_type=_F32) + b_ref[...]


@jax.jit
def _outproj(agg2d, w_o, b_o):
    return pl.pallas_call(
        _out_body,
        grid=(NPIX // _MBLK,),
        in_specs=[
            pl.BlockSpec((_MBLK, C), lambda i: (i, 0)),
            pl.BlockSpec((C, C), lambda i: (0, 0)),
            pl.BlockSpec((1, C), lambda i: (0, 0)),
        ],
        out_specs=pl.BlockSpec((_MBLK, C), lambda i: (i, 0)),
        out_shape=jax.ShapeDtypeStruct((NPIX, C), _F32),
    )(agg2d, w_o, b_o)


_CPERM = jnp.asarray(
    [(g * GC + (b // 2) * 32 + (b % 2) + 2 * j)
     for g in range(G) for b in range(4) for j in range(CP)], jnp.int32)


def kernel(x, w_v, b_v, w_om, b_om, w_o, b_o):
    x2 = x.reshape(NPIX, C)
    value, om_t = _proj(x2, w_v, w_om, b_v.reshape(1, C), b_om.reshape(OM, 1))
    agg = _dcn_sc(value.reshape(NPIX * G, GC), om_t)
    out = _outproj(agg.reshape(NPIX, C), w_o[:, _CPERM], b_o.reshape(1, C))
    return out.reshape(N, L, C)


# speedup vs baseline: 1.5386x; 1.1610x over previous
import dataclasses
import functools

import jax
import jax.numpy as jnp
from jax import lax
from jax.experimental import pallas as pl
from jax.experimental.pallas import tpu as pltpu
from jax.experimental.pallas import tpu_sc as plsc

N, H, W, C = 2, 56, 56, 512
G, GC = 8, 64
P = 9
L = H * W
OM = G * 3 * P
NPIX = N * L

NWORK = 32
CP = 16
NCHUNK = NPIX // CP
NITEM = NCHUNK * G
IPW = NITEM // NWORK
NTAP = P * 4
ROWS = NTAP * CP
IDX_MINOR = 96
IDX_MAJOR = ROWS // IDX_MINOR

_F32 = jnp.float32
_I32 = jnp.int32


def _sc_compiler_params():
    cp = pltpu.CompilerParams()
    fields = pltpu.CompilerParams.__dataclass_fields__
    if "needs_layout_passes" in fields:
        cp = dataclasses.replace(cp, needs_layout_passes=False)
    if "use_tc_tiling_on_sc" in fields:
        cp = dataclasses.replace(cp, use_tc_tiling_on_sc=False)
    return cp


def _iota16():
    return lax.iota(_I32, CP)


def _full16(v, dtype=_I32):
    return jnp.full((CP,), v, dtype)


def _floor_i32(v):
    t = v.astype(_I32)
    return jnp.where(v < t.astype(_F32), t - 1, t)


def _sc_body(val_hbm, om_hbm, out_hbm,
             omb0, omb1, idxb0, idxb1, wb0, wb1, rowsb0, rowsb1,
             outb0, outb1, oidx0, oidx1, semo, semg0, semg1, sems):
    wid = lax.axis_index("c") * 16 + lax.axis_index("s")
    base = wid * IPW
    iota = _iota16()

    def _split(item):
        chunk = item // G
        g = item - chunk * G
        return chunk * CP, g

    valt = val_hbm
    outt = out_hbm

    def om_issue(item, omb):
        p0, g = _split(item)
        pltpu.async_copy(
            om_hbm.at[pl.ds(g * (3 * P), 3 * P), pl.ds(p0, CP)], omb, semo)

    def om_wait(omb):
        pltpu.make_async_copy(
            om_hbm.at[pl.ds(0, 3 * P), pl.ds(0, CP)], omb, semo).wait()

    def stage_issue(item, omb, idxb, wb, rowsb, semg):
        p0, g = _split(item)
        pix = p0 + iota
        n = pix // L
        l = pix - n * L
        y = l // W
        x = l - y * W
        xf = x.astype(_F32)
        yf = y.astype(_F32)
        rb = n * (L * G) + g

        for t in range(P):
            offx = omb[2 * t, :]
            offy = omb[2 * t + 1, :]
            m = omb[2 * P + t, :]
            fx = xf + float(t // 3 - 1) + offx
            fy = yf + float(t % 3 - 1) + offy
            x0 = _floor_i32(fx)
            y0 = _floor_i32(fy)
            wx1 = fx - x0.astype(_F32)
            wx0 = 1.0 - wx1
            wy1 = fy - y0.astype(_F32)
            wy0 = 1.0 - wy1
            zero = jnp.zeros((CP,), _F32)
            wx0 = jnp.where((x0 >= 0) & (x0 <= W - 1), wx0, zero)
            wx1 = jnp.where((x0 >= -1) & (x0 <= W - 2), wx1, zero)
            wy0 = jnp.where((y0 >= 0) & (y0 <= H - 1), wy0, zero)
            wy1 = jnp.where((y0 >= -1) & (y0 <= H - 2), wy1, zero)
            cx0 = jnp.clip(x0, 0, W - 1)
            cx1 = jnp.clip(x0 + 1, 0, W - 1)
            cy0 = jnp.clip(y0, 0, H - 1)
            cy1 = jnp.clip(y0 + 1, 0, H - 1)
            rx0 = cx0 * G
            rx1 = cx1 * G
            ry0 = cy0 * (W * G)
            ry1 = cy1 * (W * G)
            corners = (
                (ry0 + rx0, wx0 * wy0),
                (ry0 + rx1, wx1 * wy0),
                (ry1 + rx0, wx0 * wy1),
                (ry1 + rx1, wx1 * wy1),
            )
            for ci, (radd, wgt) in enumerate(corners):
                flat = (t * 4 + ci) * CP
                idxb.at[flat // IDX_MINOR, pl.ds(flat % IDX_MINOR, CP)][...] = rb + radd
                wb.at[pl.ds(flat, CP)][...] = wgt * m

        for i in range(IDX_MAJOR):
            pltpu.async_copy(
                valt.at[idxb.at[i]],
                rowsb.at[pl.ds(i * IDX_MINOR, IDX_MINOR)], semg)

    def gather_wait(idxb, rowsb, semg):
        for i in range(IDX_MAJOR):
            pltpu.make_async_copy(
                valt.at[idxb.at[i]],
                rowsb.at[pl.ds(i * IDX_MINOR, IDX_MINOR)], semg).wait()

    def consume(item, k, wb, rowsb, outb, oidx):
        @pl.when(k >= 2)
        def _():
            pltpu.make_async_copy(outb, outt.at[oidx], sems).wait()

        p0, g = _split(item)
        pix = p0 + iota

        @pl.loop(0, CP)
        def _px(p):
            acc_a = [jnp.zeros((CP,), _F32) for _ in range(GC // CP)]
            acc_b = [jnp.zeros((CP,), _F32) for _ in range(GC // CP)]
            for tc in range(NTAP):
                wsp = plsc.load_gather(wb, [_full16(tc * CP) + p])
                row = tc * CP + p
                tgt = acc_a if tc % 2 == 0 else acc_b
                for h in range(2):
                    v2 = rowsb[row, pl.ds(h * 2 * CP, 2 * CP)]
                    ev, od = plsc.unpack(v2, format=plsc.PackFormat.INTERLEAVED)
                    tgt[2 * h] = tgt[2 * h] + wsp * ev
                    tgt[2 * h + 1] = tgt[2 * h + 1] + wsp * od
            for c in range(GC // CP):
                outb.at[p, pl.ds(c * CP, CP)][...] = acc_a[c] + acc_b[c]

        oidx[...] = pix * G + g
        pltpu.async_copy(outb, outt.at[oidx], sems)

    p0, g0 = _split(base)
    pltpu.sync_copy(om_hbm.at[pl.ds(g0 * (3 * P), 3 * P), pl.ds(p0, CP)], omb0)
    om_issue(base + 1, omb1)
    stage_issue(base, omb0, idxb0, wb0, rowsb0, semg0)

    @pl.loop(0, IPW, step=2)
    def _pair(k):
        i0 = base + k

        om_wait(omb1)

        @pl.when(k + 2 < IPW)
        def _():
            om_issue(i0 + 2, omb0)

        stage_issue(i0 + 1, omb1, idxb1, wb1, rowsb1, semg1)
        gather_wait(idxb0, rowsb0, semg0)
        consume(i0, k, wb0, rowsb0, outb0, oidx0)

        @pl.when(k + 2 < IPW)
        def _():
            om_wait(omb0)

            @pl.when(k + 3 < IPW)
            def _():
                om_issue(i0 + 3, omb1)

            stage_issue(i0 + 2, omb0, idxb0, wb0, rowsb0, semg0)

        gather_wait(idxb1, rowsb1, semg1)
        consume(i0 + 1, k + 1, wb1, rowsb1, outb1, oidx1)

    pltpu.make_async_copy(outb0, outt.at[oidx0], sems).wait()
    pltpu.make_async_copy(outb1, outt.at[oidx1], sems).wait()


@jax.jit
def _dcn_sc(val2d, om2d):
    mesh = plsc.VectorSubcoreMesh(core_axis_name="c", subcore_axis_name="s")
    f = pl.kernel(
        _sc_body,
        out_type=jax.ShapeDtypeStruct((NPIX * G, GC), _F32),
        mesh=mesh,
        scratch_types=[
            pltpu.VMEM((3 * P, CP), _F32),
            pltpu.VMEM((3 * P, CP), _F32),
            pltpu.VMEM((IDX_MAJOR, IDX_MINOR), _I32),
            pltpu.VMEM((IDX_MAJOR, IDX_MINOR), _I32),
            pltpu.VMEM((ROWS,), _F32),
            pltpu.VMEM((ROWS,), _F32),
            pltpu.VMEM((ROWS, GC), jnp.bfloat16),
            pltpu.VMEM((ROWS, GC), jnp.bfloat16),
            pltpu.VMEM((CP, GC), _F32),
            pltpu.VMEM((CP, GC), _F32),
            pltpu.VMEM((CP,), _I32),
            pltpu.VMEM((CP,), _I32),
            pltpu.SemaphoreType.DMA,
            pltpu.SemaphoreType.DMA,
            pltpu.SemaphoreType.DMA,
            pltpu.SemaphoreType.DMA,
        ],
        compiler_params=_sc_compiler_params(),
    )
    return f(val2d, om2d)


_MBLK = 896


def _proj_body(x_ref, wv_ref, wom_ref, bv_ref, bom_ref, val_ref, om_ref):
    xb = x_ref[...]
    dn = (((1,), (1,)), ((), ()))
    xb16 = xb.astype(jnp.bfloat16)
    val_ref[...] = (lax.dot_general(
        xb16, wv_ref[...].astype(jnp.bfloat16), dn,
        preferred_element_type=_F32) + bv_ref[...]).astype(jnp.bfloat16)
    om_ref[...] = lax.dot_general(
        wom_ref[...], xb, dn,
        preferred_element_type=_F32) + bom_ref[...]


@jax.jit
def _proj(x2d, w_v, w_om, b_v, b_om):
    grid = (NPIX // _MBLK,)
    return pl.pallas_call(
        _proj_body,
        grid=grid,
        in_specs=[
            pl.BlockSpec((_MBLK, C), lambda i: (i, 0)),
            pl.BlockSpec((C, C), lambda i: (0, 0)),
            pl.BlockSpec((OM, C), lambda i: (0, 0)),
            pl.BlockSpec((1, C), lambda i: (0, 0)),
            pl.BlockSpec((OM, 1), lambda i: (0, 0)),
        ],
        out_specs=[
            pl.BlockSpec((_MBLK, C), lambda i: (i, 0)),
            pl.BlockSpec((OM, _MBLK), lambda i: (0, i)),
        ],
        out_shape=[
            jax.ShapeDtypeStruct((NPIX, C), jnp.bfloat16),
            jax.ShapeDtypeStruct((OM, NPIX), _F32),
        ],
    )(x2d, w_v, w_om, b_v, b_om)


def _out_body(a_ref, w_ref, b_ref, o_ref):
    o_ref[...] = lax.dot_general(
        a_ref[...].astype(jnp.bfloat16), w_ref[...].astype(jnp.bfloat16),
        (((1,), (1,)), ((), ())),
        preferred_element_type=_F32) + b_ref[...]


_OBLK = 392


@jax.jit
def _outproj(agg2d, w_o, b_o):
    return pl.pallas_call(
        _out_body,
        grid=(NPIX // _OBLK,),
        in_specs=[
            pl.BlockSpec((_OBLK, C), lambda i: (i, 0)),
            pl.BlockSpec((C, C), lambda i: (0, 0)),
            pl.BlockSpec((1, C), lambda i: (0, 0)),
        ],
        out_specs=pl.BlockSpec((_OBLK, C), lambda i: (i, 0)),
        out_shape=jax.ShapeDtypeStruct((NPIX, C), _F32),
    )(agg2d, w_o, b_o)


_CPERM = jnp.asarray(
    [(g * GC + (b // 2) * 32 + (b % 2) + 2 * j)
     for g in range(G) for b in range(4) for j in range(CP)], jnp.int32)


def kernel(x, w_v, b_v, w_om, b_om, w_o, b_o):
    x2 = x.reshape(NPIX, C)
    value, om_t = _proj(x2, w_v, w_om, b_v.reshape(1, C), b_om.reshape(OM, 1))
    agg = _dcn_sc(value.reshape(NPIX * G, GC), om_t)
    out = _outproj(agg.reshape(NPIX, C), w_o[:, _CPERM], b_o.reshape(1, C))
    return out.reshape(N, L, C)
